# Initial kernel scaffold; baseline (speedup 1.0000x reference)
import jax
import jax.numpy as jnp
from jax.experimental import pallas as pl


def kernel(in_degree, out_degree, table1, table2, W, b):
    raise NotImplementedError("write your pallas kernel here")


# speedup vs baseline: 2.5500x; 2.5500x over previous
import functools

import jax
import jax.numpy as jnp
from jax import lax
from jax.experimental import pallas as pl
from jax.experimental.pallas import tpu as pltpu
from jax.experimental.pallas import tpu_sc as plsc

MAX_DEG = 512
ROWS_PAD = 520
D_IN = 128
D_OUT = 64
L = 16
CH = 128


def _project_body(t1_ref, t2_ref, w_ref, b_ref, p1_ref, p2_ref):
    w = w_ref[...]
    dn = (((1,), (1,)), ((), ()))
    p1_ref[...] = lax.dot_general(t1_ref[...], w, dn,
                                  preferred_element_type=jnp.float32)
    p2_ref[...] = lax.dot_general(t2_ref[...], w, dn,
                                  preferred_element_type=jnp.float32) + b_ref[...]


def _make_sc_kernel(n_total):
    nc, ns = 2, 16
    nw = nc * ns
    cnt = (-(-n_total // nw) + 7) // 8 * 8
    assert n_total % 8 == 0 and cnt <= n_total
    k_chunks = -(-cnt // CH)

    mesh = plsc.VectorSubcoreMesh(core_axis_name="c", subcore_axis_name="s",
                                  num_cores=nc, num_subcores=ns)

    @functools.partial(
        pl.kernel,
        out_type=jax.ShapeDtypeStruct((n_total, D_OUT), jnp.float32),
        mesh=mesh,
        scratch_types=[
            pltpu.VMEM((CH,), jnp.int32),
            pltpu.VMEM((CH,), jnp.int32),
            pltpu.VMEM((CH, D_OUT), jnp.float32),
            pltpu.VMEM((CH, D_OUT), jnp.float32),
            pltpu.SemaphoreType.DMA,
            pltpu.SemaphoreType.DMA,
        ],
        compiler_params=pltpu.CompilerParams(use_tc_tiling_on_sc=False),
    )
    def sc_kernel(t1p_hbm, t2p_hbm, ind_hbm, outd_hbm, out_hbm,
                  idx1_v, idx2_v, rows1_v, rows2_v, sem1, sem2):
        wid = lax.axis_index("s") * nc + lax.axis_index("c")
        base = jnp.minimum(wid * cnt, n_total - cnt)

        def chunk_body(c, carry):
            start = base + jnp.minimum(c * CH, cnt - CH)
            pltpu.sync_copy(ind_hbm.at[pl.ds(start, CH)], idx1_v)
            pltpu.sync_copy(outd_hbm.at[pl.ds(start, CH)], idx2_v)

            def clamp_body(j, carry2):
                s = pl.ds(j * L, L)
                idx1_v[s] = jnp.clip(idx1_v[s], 0, MAX_DEG)
                idx2_v[s] = jnp.clip(idx2_v[s], 0, MAX_DEG)
                return carry2

            lax.fori_loop(0, CH // L, clamp_body, 0)

            cp1 = pltpu.async_copy(t1p_hbm.at[idx1_v], rows1_v, sem1)
            cp2 = pltpu.async_copy(t2p_hbm.at[idx2_v], rows2_v, sem2)
            cp1.wait()
            cp2.wait()

            def add_body(r, carry2):
                for jj in range(D_OUT // L):
                    s = pl.ds(jj * L, L)
                    plsc.addupdate(rows1_v.at[r, s], rows2_v[r, s])
                return carry2

            lax.fori_loop(0, CH, add_body, 0)

            pltpu.sync_copy(rows1_v, out_hbm.at[pl.ds(start, CH)])
            return carry

        lax.fori_loop(0, k_chunks, chunk_body, 0)

    return sc_kernel


def kernel(in_degree, out_degree, table1, table2, W, b):
    n_total = in_degree.shape[0]
    pad = ROWS_PAD - table1.shape[0]
    t1 = jnp.pad(table1, ((0, pad), (0, 0)))
    t2 = jnp.pad(table2, ((0, pad), (0, 0)))
    b2 = b.reshape(1, D_OUT)

    t1p, t2p = pl.pallas_call(
        _project_body,
        out_shape=[jax.ShapeDtypeStruct((ROWS_PAD, D_OUT), jnp.float32)] * 2,
    )(t1, t2, W, b2)

    sc_kernel = _make_sc_kernel(n_total)
    return sc_kernel(t1p, t2p,
                     in_degree.astype(jnp.int32),
                     out_degree.astype(jnp.int32))


# speedup vs baseline: 2.9808x; 1.1689x over previous
import functools

import jax
import jax.numpy as jnp
from jax import lax
from jax.experimental import pallas as pl
from jax.experimental.pallas import tpu as pltpu
from jax.experimental.pallas import tpu_sc as plsc

MAX_DEG = 512
ROWS_PAD = 520
D_IN = 128
D_OUT = 64
L = 16
CH = 128


def _project_body(t1_ref, t2_ref, w_ref, b_ref, p1_ref, p2_ref):
    w = w_ref[...]
    dn = (((1,), (1,)), ((), ()))
    p1_ref[...] = lax.dot_general(t1_ref[...], w, dn,
                                  preferred_element_type=jnp.float32)
    p2_ref[...] = lax.dot_general(t2_ref[...], w, dn,
                                  preferred_element_type=jnp.float32) + b_ref[...]


def _make_sc_kernel(n_total):
    nc, ns = 2, 16
    nw = nc * ns
    cnt = (-(-n_total // nw) + 7) // 8 * 8
    assert n_total % 8 == 0 and cnt <= n_total
    k_chunks = -(-cnt // CH)

    mesh = plsc.VectorSubcoreMesh(core_axis_name="c", subcore_axis_name="s",
                                  num_cores=nc, num_subcores=ns)
    NB = 3

    @functools.partial(
        pl.kernel,
        out_type=jax.ShapeDtypeStruct((n_total, D_OUT), jnp.float32),
        mesh=mesh,
        scratch_types=[
            pltpu.VMEM((NB, CH), jnp.int32),
            pltpu.VMEM((NB, CH), jnp.int32),
            pltpu.VMEM((NB, CH, D_OUT), jnp.float32),
            pltpu.VMEM((NB, CH, D_OUT), jnp.float32),
            [pltpu.SemaphoreType.DMA] * NB,
            [pltpu.SemaphoreType.DMA] * NB,
            [pltpu.SemaphoreType.DMA] * NB,
        ],
        compiler_params=pltpu.CompilerParams(use_tc_tiling_on_sc=False),
    )
    def sc_kernel(t1p_hbm, t2p_hbm, ind_hbm, outd_hbm, out_hbm,
                  idx1_v, idx2_v, rows1_v, rows2_v,
                  sem_idx, sem_g, sem_out):
        wid = lax.axis_index("s") * nc + lax.axis_index("c")
        base = jnp.minimum(wid * cnt, n_total - cnt)
        starts = [None] * k_chunks
        cp_idx = [None] * k_chunks
        cp_g = [None] * k_chunks
        cp_out = [None] * k_chunks

        def fire_idx(c):
            b = c % NB
            starts[c] = base + min(c * CH, cnt - CH)
            s = pl.ds(starts[c], CH)
            cp_idx[c] = (
                pltpu.async_copy(ind_hbm.at[s], idx1_v.at[b], sem_idx[b]),
                pltpu.async_copy(outd_hbm.at[s], idx2_v.at[b], sem_idx[b]),
            )

        def fire_gather(c):
            b = c % NB
            cp_idx[c][0].wait()
            cp_idx[c][1].wait()
            for j in range(CH // L):
                s = pl.ds(j * L, L)
                idx1_v[b, s] = jnp.clip(idx1_v[b, s], 0, MAX_DEG)
                idx2_v[b, s] = jnp.clip(idx2_v[b, s], 0, MAX_DEG)
            if c >= NB:
                cp_out[c - NB].wait()
            cp_g[c] = (
                pltpu.async_copy(t1p_hbm.at[idx1_v.at[b]], rows1_v.at[b],
                                 sem_g[b]),
                pltpu.async_copy(t2p_hbm.at[idx2_v.at[b]], rows2_v.at[b],
                                 sem_g[b]),
            )

        def add_and_out(c):
            b = c % NB
            cp_g[c][0].wait()
            cp_g[c][1].wait()

            def add_body(j, carry):
                for k in range(16):
                    r = j * 4 + k // 4
                    col = pl.ds((k % 4) * L, L)
                    plsc.addupdate(rows1_v.at[b, r, col], rows2_v[b, r, col])
                return carry

            lax.fori_loop(0, CH // 4, add_body, 0, unroll=False)
            cp_out[c] = pltpu.async_copy(rows1_v.at[b],
                                         out_hbm.at[pl.ds(starts[c], CH)],
                                         sem_out[b])

        fire_idx(0)
        fire_idx(1)
        fire_gather(0)
        for c in range(k_chunks):
            if c + 2 < k_chunks:
                fire_idx(c + 2)
            if c + 1 < k_chunks:
                fire_gather(c + 1)
            add_and_out(c)
        for c in range(max(0, k_chunks - NB), k_chunks):
            cp_out[c].wait()

    return sc_kernel


def kernel(in_degree, out_degree, table1, table2, W, b):
    n_total = in_degree.shape[0]
    pad = ROWS_PAD - table1.shape[0]
    t1 = jnp.pad(table1, ((0, pad), (0, 0)))
    t2 = jnp.pad(table2, ((0, pad), (0, 0)))
    b2 = b.reshape(1, D_OUT)

    t1p, t2p = pl.pallas_call(
        _project_body,
        out_shape=[jax.ShapeDtypeStruct((ROWS_PAD, D_OUT), jnp.float32)] * 2,
    )(t1, t2, W, b2)

    sc_kernel = _make_sc_kernel(n_total)
    return sc_kernel(t1p, t2p,
                     in_degree.astype(jnp.int32),
                     out_degree.astype(jnp.int32))
